# baseline (speedup 1.0000x reference)
import functools

import jax
import jax.numpy as jnp
from jax.experimental import pallas as pl
from jax.experimental.pallas import tpu as pltpu

_TH = 32
_CG = 4


def _shift_w(v, k):
    z = jnp.zeros(v.shape[:-1] + (1,), v.dtype)
    if k == -1:
        return jnp.concatenate([z, v[..., :-1]], axis=-1)
    return jnp.concatenate([v[..., 1:], z], axis=-1)


def _body(xm_ref, xt_ref, xb_ref, w_ref, p_ref, t_ref, s_ref, o_ref,
          xe_ref, agg_ref, h1b_ref):
    th = o_ref.shape[2]
    i = pl.program_id(1)
    ni = pl.num_programs(1)
    bf = jnp.bfloat16
    zrow = jnp.zeros((96, 1, 224), jnp.float32)

    xe_ref[:, 8:8 + th] = xm_ref[0]
    xe_ref[:, 7:8] = jnp.where(i > 0, xt_ref[0, :, 7:8, :], zrow)
    xe_ref[:, 8 + th:9 + th] = jnp.where(i < ni - 1, xb_ref[0, :, 0:1, :], zrow)

    p = [p_ref[k].astype(bf) for k in range(8)]
    acc_a1 = jnp.zeros((th, 224), jnp.float32)
    acc_cross = jnp.zeros((th, 224), jnp.float32)

    for c0 in range(0, 96, _CG):
        h0 = xe_ref[c0:c0 + _CG, 7:7 + th].astype(bf)
        h1 = xe_ref[c0:c0 + _CG, 8:8 + th].astype(bf)
        h2 = xe_ref[c0:c0 + _CG, 9:9 + th].astype(bf)

        acc_a1 = acc_a1 + jnp.sum(h1 * h1, axis=0)
        hs = h0 + h1 + h2
        bx = _shift_w(hs, -1) + hs + _shift_w(hs, 1)
        acc_cross = acc_cross + jnp.sum(h1 * bx, axis=0)

        cl = p[0] * h0 + p[3] * h1 + p[5] * h2
        cm = p[1] * h0 + p[6] * h2
        cr = p[2] * h0 + p[4] * h1 + p[7] * h2
        agg_ref[c0:c0 + _CG] = _shift_w(cl, -1) + cm + _shift_w(cr, 1)
        h1b_ref[c0:c0 + _CG] = h1

    trow = xe_ref[:, 7, :]
    brow = xe_ref[:, 8 + th, :]
    atop = jnp.sum(trow * trow, axis=0)[None]
    abot = jnp.sum(brow * brow, axis=0)[None]
    a0 = jnp.concatenate([atop, acc_a1[:th - 1]], axis=0)
    a2 = jnp.concatenate([acc_a1[1:], abot], axis=0)
    ah = a0 + acc_a1 + a2
    a9 = _shift_w(ah, -1) + ah + _shift_w(ah, 1)

    div = a9 - 2.0 * acc_cross + 9.0 * acc_a1
    z = (div - t_ref[0]) * s_ref[0]
    mask = z > 0.0

    sel = jnp.where(mask[None], agg_ref[...], h1b_ref[...])
    o = jnp.dot(w_ref[...], sel.reshape(96, th * 224),
                preferred_element_type=jnp.float32)
    o_ref[0] = o.reshape(96, th, 224)


@functools.partial(jax.jit, static_argnames=())
def kernel(x, core, periphery, threshold, scale):
    B, C, H, W = x.shape
    O = core.shape[0]
    w = core.reshape(O, C).astype(jnp.bfloat16)
    nh = H // _TH
    nh8 = H // 8

    out = pl.pallas_call(
        _body,
        grid=(B, nh),
        in_specs=[
            pl.BlockSpec((1, C, _TH, W), lambda b, i: (b, 0, i, 0)),
            pl.BlockSpec((1, C, 8, W),
                         lambda b, i: (b, 0, jnp.maximum(i * (_TH // 8) - 1, 0), 0)),
            pl.BlockSpec((1, C, 8, W),
                         lambda b, i: (b, 0,
                                       jnp.minimum(i * (_TH // 8) + _TH // 8,
                                                   nh8 - 1), 0)),
            pl.BlockSpec((O, C), lambda b, i: (0, 0)),
            pl.BlockSpec(memory_space=pltpu.SMEM),
            pl.BlockSpec(memory_space=pltpu.SMEM),
            pl.BlockSpec(memory_space=pltpu.SMEM),
        ],
        out_specs=pl.BlockSpec((1, O, _TH, W), lambda b, i: (b, 0, i, 0)),
        out_shape=jax.ShapeDtypeStruct((B, O, H, W), jnp.float32),
        scratch_shapes=[
            pltpu.VMEM((C, _TH + 16, W), jnp.float32),
            pltpu.VMEM((C, _TH, W), jnp.bfloat16),
            pltpu.VMEM((C, _TH, W), jnp.bfloat16),
        ],
    )(x, x, x, w, periphery, threshold, scale)
    return out


# speedup vs baseline: 1.7000x; 1.7000x over previous
import functools

import jax
import jax.numpy as jnp
from jax.experimental import pallas as pl
from jax.experimental.pallas import tpu as pltpu

_TH = 32
_CG = 4


def _shift_w(v, k):
    z = jnp.zeros(v.shape[:-1] + (1,), v.dtype)
    if k == -1:
        return jnp.concatenate([z, v[..., :-1]], axis=-1)
    return jnp.concatenate([v[..., 1:], z], axis=-1)


def _body(xm_ref, xt_ref, xb_ref, w_ref, p_ref, t_ref, s_ref, o_ref,
          agg_ref, h1b_ref):
    th = o_ref.shape[2]
    i = pl.program_id(1)
    ni = pl.num_programs(1)
    bf = jnp.bfloat16
    zrow = jnp.zeros((96, 1, 224), jnp.float32)
    top = jnp.where(i > 0, xt_ref[0, :, 7:8, :], zrow)
    bot = jnp.where(i < ni - 1, xb_ref[0, :, 0:1, :], zrow)

    p = [p_ref[k].astype(bf) for k in range(8)]
    acc_a1 = jnp.zeros((th, 224), jnp.float32)
    acc_cross = jnp.zeros((th, 224), jnp.float32)

    for c0 in range(0, 96, _CG):
        cs = slice(c0, c0 + _CG)
        h1f = xm_ref[0, cs]
        h0 = jnp.concatenate([top[cs], h1f[:, :th - 1]], axis=1).astype(bf)
        h2 = jnp.concatenate([h1f[:, 1:], bot[cs]], axis=1).astype(bf)
        h1 = h1f.astype(bf)

        acc_a1 = acc_a1 + jnp.sum(h1 * h1, axis=0)
        hs = h0 + h1 + h2
        bx = _shift_w(hs, -1) + hs + _shift_w(hs, 1)
        acc_cross = acc_cross + jnp.sum(h1 * bx, axis=0)

        cl = p[0] * h0 + p[3] * h1 + p[5] * h2
        cm = p[1] * h0 + p[6] * h2
        cr = p[2] * h0 + p[4] * h1 + p[7] * h2
        agg_ref[cs] = _shift_w(cl, -1) + cm + _shift_w(cr, 1)
        h1b_ref[cs] = h1

    atop = jnp.sum(top[:, 0] * top[:, 0], axis=0)[None]
    abot = jnp.sum(bot[:, 0] * bot[:, 0], axis=0)[None]
    a0 = jnp.concatenate([atop, acc_a1[:th - 1]], axis=0)
    a2 = jnp.concatenate([acc_a1[1:], abot], axis=0)
    ah = a0 + acc_a1 + a2
    a9 = _shift_w(ah, -1) + ah + _shift_w(ah, 1)

    div = a9 - 2.0 * acc_cross + 9.0 * acc_a1
    z = (div - t_ref[0]) * s_ref[0]
    mask = z > 0.0

    sel = jnp.where(mask[None], agg_ref[...], h1b_ref[...])
    o = jnp.dot(w_ref[...], sel.reshape(96, th * 224),
                preferred_element_type=jnp.float32)
    o_ref[0] = o.reshape(96, th, 224)


@functools.partial(jax.jit, static_argnames=())
def kernel(x, core, periphery, threshold, scale):
    B, C, H, W = x.shape
    O = core.shape[0]
    w = core.reshape(O, C).astype(jnp.bfloat16)
    nh = H // _TH
    nh8 = H // 8

    out = pl.pallas_call(
        _body,
        grid=(B, nh),
        in_specs=[
            pl.BlockSpec((1, C, _TH, W), lambda b, i: (b, 0, i, 0)),
            pl.BlockSpec((1, C, 8, W),
                         lambda b, i: (b, 0, jnp.maximum(i * (_TH // 8) - 1, 0), 0)),
            pl.BlockSpec((1, C, 8, W),
                         lambda b, i: (b, 0,
                                       jnp.minimum(i * (_TH // 8) + _TH // 8,
                                                   nh8 - 1), 0)),
            pl.BlockSpec((O, C), lambda b, i: (0, 0)),
            pl.BlockSpec(memory_space=pltpu.SMEM),
            pl.BlockSpec(memory_space=pltpu.SMEM),
            pl.BlockSpec(memory_space=pltpu.SMEM),
        ],
        out_specs=pl.BlockSpec((1, O, _TH, W), lambda b, i: (b, 0, i, 0)),
        out_shape=jax.ShapeDtypeStruct((B, O, H, W), jnp.float32),
        scratch_shapes=[
            pltpu.VMEM((C, _TH, W), jnp.bfloat16),
            pltpu.VMEM((C, _TH, W), jnp.bfloat16),
        ],
    )(x, x, x, w, periphery, threshold, scale)
    return out
